# Initial kernel scaffold; baseline (speedup 1.0000x reference)
import jax
import jax.numpy as jnp
from jax.experimental import pallas as pl


def kernel(x, edge_index, W1, b1, g1, be1, W2, b2, g2, be2):
    raise NotImplementedError("write your pallas kernel here")


# speedup vs baseline: 6.6283x; 6.6283x over previous
import functools

import jax
import jax.numpy as jnp
from jax import lax
from jax.experimental import pallas as pl
from jax.experimental.pallas import tpu as pltpu
from jax.experimental.pallas import tpu_sc as plsc

_N = 10000
_E = 320000
_D = 128

_NC = 2
_NS = 16
_NW = _NC * _NS
_CH = 80
_EPW = _E // _NW
_NCHT = _EPW // _CH
_RPT = 624
_RREM = _N - _RPT * _NS

_mesh = plsc.VectorSubcoreMesh(core_axis_name="c", subcore_axis_name="s")


@functools.partial(
    pl.kernel,
    mesh=_mesh,
    out_type=jax.ShapeDtypeStruct((_NC, _N, _D), jnp.float32),
    scratch_types=[
        pltpu.VMEM((_NCHT, _CH), jnp.int32),
        pltpu.VMEM((_NCHT, _CH), jnp.int32),
        pltpu.VMEM((_CH, _D), jnp.float32),
        pltpu.VMEM_SHARED((_N, _D), jnp.float32),
        pltpu.SemaphoreType.DMA,
    ],
)
def _sc_agg(x_hbm, src_hbm, dst_hbm, z_hbm, out_hbm, src_v, dst_v, rows_v,
            agg_sh, sem):
    c = lax.axis_index("c")
    s = lax.axis_index("s")
    wid = s * _NC + c
    pltpu.sync_copy(z_hbm.at[pl.ds(s * _RPT, _RPT)],
                    agg_sh.at[pl.ds(s * _RPT, _RPT)])

    @pl.when(s == 0)
    def _zero_tail():
        pltpu.sync_copy(z_hbm.at[pl.ds(_RPT * _NS, _RREM)],
                        agg_sh.at[pl.ds(_RPT * _NS, _RREM)])
    pltpu.sync_copy(src_hbm.at[wid], src_v)
    pltpu.sync_copy(dst_hbm.at[wid], dst_v)
    plsc.subcore_barrier()

    def body(j, carry):
        pltpu.async_copy(x_hbm.at[src_v.at[j]], rows_v, sem).wait()
        pltpu.sync_copy(rows_v, agg_sh.at[dst_v.at[j]], add=True)
        return carry

    lax.fori_loop(0, _NCHT, body, 0)
    plsc.subcore_barrier()
    pltpu.sync_copy(agg_sh.at[pl.ds(s * _RPT, _RPT)],
                    out_hbm.at[c, pl.ds(s * _RPT, _RPT)])

    @pl.when(s == 0)
    def _write_tail():
        pltpu.sync_copy(agg_sh.at[pl.ds(_RPT * _NS, _RREM)],
                        out_hbm.at[c, pl.ds(_RPT * _NS, _RREM)])


def _mlp_body(x_ref, agg_ref, w_ref, b_ref, g_ref, be_ref, out_ref):
    h = x_ref[...] + agg_ref[0] + agg_ref[1]
    t = lax.dot_general(h, w_ref[...], (((1,), (1,)), ((), ())),
                        preferred_element_type=jnp.float32)
    t = jnp.maximum(t + b_ref[...], 0.0)
    mean = jnp.mean(t, axis=0, keepdims=True)
    ctr = t - mean
    var = jnp.mean(ctr * ctr, axis=0, keepdims=True)
    out_ref[...] = ctr * lax.rsqrt(var + 1e-5) * g_ref[...] + be_ref[...]


def _mlp(x, agg, w, b, g, be):
    return pl.pallas_call(
        _mlp_body,
        out_shape=jax.ShapeDtypeStruct((_N, _D), jnp.float32),
    )(x, agg, w, b.reshape(1, _D), g.reshape(1, _D), be.reshape(1, _D))


def kernel(x, edge_index, W1, b1, g1, be1, W2, b2, g2, be2):
    src = edge_index[0].astype(jnp.int32).reshape(_NW, _NCHT, _CH)
    dst = edge_index[1].astype(jnp.int32).reshape(_NW, _NCHT, _CH)
    z = jnp.zeros((_N, _D), jnp.float32)
    agg1 = _sc_agg(x, src, dst, z)
    h1 = _mlp(x, agg1, W1, b1, g1, be1)
    agg2 = _sc_agg(h1, src, dst, z)
    h2 = _mlp(h1, agg2, W2, b2, g2, be2)
    return h2
